# Initial kernel scaffold; baseline (speedup 1.0000x reference)
import jax
import jax.numpy as jnp
from jax.experimental import pallas as pl


def kernel(x, edge_index, W, b):
    raise NotImplementedError("write your pallas kernel here")


# speedup vs baseline: 4.7122x; 4.7122x over previous
import functools

import jax
import jax.numpy as jnp
from jax import lax
from jax.experimental import pallas as pl
from jax.experimental.pallas import tpu as pltpu
from jax.experimental.pallas import tpu_sc as plsc

_NC = 2
_NS = 16
_NW = _NC * _NS
_K = 80
_L = 16
_TRASH = 8


def _sc_body(n_chunks, half, rpt, src_hbm, dst_hbm, x_hbm, zeros_hbm,
             out_hbm, acc0, acc1, sidx, didx, idx0, idx1, rows, sem):
    c = lax.axis_index("c")
    s = lax.axis_index("s")
    wid = s * _NC + c

    row0 = s * rpt
    pltpu.sync_copy(zeros_hbm.at[pl.ds(row0, rpt)], acc0.at[pl.ds(row0, rpt)])
    pltpu.sync_copy(zeros_hbm.at[pl.ds(row0, rpt)], acc1.at[pl.ds(row0, rpt)])
    plsc.subcore_barrier()

    base = wid * n_chunks * _K

    def chunk(i, _):
        off = pl.multiple_of(base + i * _K, 8)
        pltpu.sync_copy(src_hbm.at[pl.ds(off, _K)], sidx)
        pltpu.sync_copy(dst_hbm.at[pl.ds(off, _K)], didx)
        pltpu.async_copy(x_hbm.at[sidx], rows, sem).wait()
        for g in range(_K // _L):
            dv = didx[pl.ds(g * _L, _L)]
            lo = dv < half
            idx0[pl.ds(g * _L, _L)] = jnp.where(lo, dv + _TRASH, 0)
            idx1[pl.ds(g * _L, _L)] = jnp.where(lo, 0, dv - half + _TRASH)
        pltpu.sync_copy(rows, acc0.at[idx0], add=True)
        pltpu.sync_copy(rows, acc1.at[idx1], add=True)
        return 0

    lax.fori_loop(0, n_chunks, chunk, 0)

    plsc.subcore_barrier()
    wpt = half // _NS
    wrow = s * wpt
    pltpu.sync_copy(acc0.at[pl.ds(_TRASH + wrow, wpt)],
                    out_hbm.at[c, pl.ds(wrow, wpt)])
    pltpu.sync_copy(acc1.at[pl.ds(_TRASH + wrow, wpt)],
                    out_hbm.at[c, pl.ds(half + wrow, wpt)])


def _tc_linear_body(a_ref, w_ref, b_ref, o_ref):
    acc = a_ref[0] + a_ref[1]
    o_ref[...] = lax.dot_general(
        acc, w_ref[...], (((1,), (1,)), ((), ())),
        preferred_element_type=jnp.float32) + b_ref[...]


def kernel(x, edge_index, W, b):
    n, d = x.shape
    e = edge_index.shape[1]
    c_out = W.shape[0]
    assert e % (_NW * _K) == 0
    n_chunks = e // (_NW * _K)
    half = -(-n // (2 * _NS * 8)) * (_NS * 8)
    n_pad = 2 * half
    rpt = -(-(half + _TRASH) // (_NS * 8)) * 8
    acc_rows = rpt * _NS

    src = edge_index[0]
    dst = edge_index[1]
    zeros = jnp.zeros((acc_rows, d), jnp.float32)

    sc_agg = pl.kernel(
        functools.partial(_sc_body, n_chunks, half, rpt),
        out_type=jax.ShapeDtypeStruct((_NC, n_pad, d), jnp.float32),
        mesh=plsc.VectorSubcoreMesh(core_axis_name="c", subcore_axis_name="s"),
        scratch_types=[
            pltpu.VMEM_SHARED((acc_rows, d), jnp.float32),
            pltpu.VMEM_SHARED((acc_rows, d), jnp.float32),
            pltpu.VMEM((_K,), jnp.int32),
            pltpu.VMEM((_K,), jnp.int32),
            pltpu.VMEM((_K,), jnp.int32),
            pltpu.VMEM((_K,), jnp.int32),
            pltpu.VMEM((_K, d), jnp.float32),
            pltpu.SemaphoreType.DMA,
        ],
    )
    agg2 = sc_agg(src, dst, x, zeros)

    bn = 2048
    out = pl.pallas_call(
        _tc_linear_body,
        grid=(n_pad // bn,),
        in_specs=[
            pl.BlockSpec((_NC, bn, d), lambda i: (0, i, 0)),
            pl.BlockSpec((c_out, d), lambda i: (0, 0)),
            pl.BlockSpec((1, c_out), lambda i: (0, 0)),
        ],
        out_specs=pl.BlockSpec((bn, c_out), lambda i: (i, 0)),
        out_shape=jax.ShapeDtypeStruct((n_pad, c_out), jnp.float32),
    )(agg2, W, b.reshape(1, c_out))
    return out[:n]


# speedup vs baseline: 6.5107x; 1.3817x over previous
import functools

import jax
import jax.numpy as jnp
from jax import lax
from jax.experimental import pallas as pl
from jax.experimental.pallas import tpu as pltpu
from jax.experimental.pallas import tpu_sc as plsc

_NC = 2
_NS = 16
_NW = _NC * _NS
_K = 80
_L = 16
_TRASH = 8


def _sc_body(n_chunks, half, rpt, src_hbm, dst_hbm, x_hbm, zeros_hbm,
             out_hbm, acc0, acc1, sall, dall, idx0a, idx1a, idx0b, idx1b,
             rows0, rows1, gsem0, gsem1, ssem0, ssem1):
    c = lax.axis_index("c")
    s = lax.axis_index("s")
    wid = s * _NC + c
    ept = n_chunks * _K
    base = wid * ept

    pltpu.sync_copy(src_hbm.at[pl.ds(base, ept)], sall)
    pltpu.sync_copy(dst_hbm.at[pl.ds(base, ept)], dall)

    row0 = s * rpt
    pltpu.sync_copy(zeros_hbm.at[pl.ds(row0, rpt)], acc0.at[pl.ds(row0, rpt)])
    pltpu.sync_copy(zeros_hbm.at[pl.ds(row0, rpt)], acc1.at[pl.ds(row0, rpt)])
    plsc.subcore_barrier()

    def fire_gather(ch, rows, gsem):
        off = jnp.minimum(ch, n_chunks - 1) * _K
        pltpu.async_copy(x_hbm.at[sall.at[pl.ds(off, _K)]], rows, gsem)

    def wait_gather(rows, gsem):
        pltpu.make_async_copy(x_hbm.at[pl.ds(0, _K)], rows, gsem).wait()

    def fire_scatter(ch, rows, idx0, idx1, ssem):
        valid = jnp.where(ch < n_chunks, 1, 0).astype(jnp.int32)
        off = jnp.minimum(ch, n_chunks - 1) * _K
        for g in range(_K // _L):
            dv = dall[pl.ds(off + g * _L, _L)]
            lo = dv < half
            idx0[pl.ds(g * _L, _L)] = jnp.where(lo, dv + _TRASH, 0) * valid
            idx1[pl.ds(g * _L, _L)] = jnp.where(lo, 0, dv - half + _TRASH) * valid
        pltpu.async_copy(rows, acc0.at[idx0], ssem, add=True)
        pltpu.async_copy(rows, acc1.at[idx1], ssem, add=True)

    def wait_scatters(rows, ssem):
        pltpu.make_async_copy(x_hbm.at[pl.ds(0, _K)], rows, ssem).wait()
        pltpu.make_async_copy(x_hbm.at[pl.ds(0, _K)], rows, ssem).wait()

    fire_gather(0, rows0, gsem0)
    fire_gather(1, rows1, gsem1)

    def pair(o, _):
        c0 = 2 * o
        wait_gather(rows0, gsem0)
        fire_scatter(c0, rows0, idx0a, idx1a, ssem0)
        wait_gather(rows1, gsem1)
        fire_scatter(c0 + 1, rows1, idx0b, idx1b, ssem1)
        wait_scatters(rows0, ssem0)
        fire_gather(c0 + 2, rows0, gsem0)
        wait_scatters(rows1, ssem1)
        fire_gather(c0 + 3, rows1, gsem1)
        return 0

    n_pairs = (n_chunks + 1) // 2
    lax.fori_loop(0, n_pairs, pair, 0)
    wait_gather(rows0, gsem0)
    wait_gather(rows1, gsem1)

    plsc.subcore_barrier()
    wpt = half // _NS
    wrow = s * wpt
    pltpu.sync_copy(acc0.at[pl.ds(_TRASH + wrow, wpt)],
                    out_hbm.at[c, pl.ds(wrow, wpt)])
    pltpu.sync_copy(acc1.at[pl.ds(_TRASH + wrow, wpt)],
                    out_hbm.at[c, pl.ds(half + wrow, wpt)])


def _tc_linear_body(a_ref, w_ref, b_ref, o_ref):
    acc = a_ref[0] + a_ref[1]
    o_ref[...] = lax.dot_general(
        acc, w_ref[...], (((1,), (1,)), ((), ())),
        preferred_element_type=jnp.float32) + b_ref[...]


def kernel(x, edge_index, W, b):
    n, d = x.shape
    e = edge_index.shape[1]
    c_out = W.shape[0]
    assert e % (_NW * _K) == 0
    n_chunks = e // (_NW * _K)
    half = -(-n // (2 * _NS * 8)) * (_NS * 8)
    n_pad = 2 * half
    rpt = -(-(half + _TRASH) // (_NS * 8)) * 8
    acc_rows = rpt * _NS
    ept = n_chunks * _K

    src = edge_index[0]
    dst = edge_index[1]
    zeros = jnp.zeros((acc_rows, d), jnp.float32)

    sc_agg = pl.kernel(
        functools.partial(_sc_body, n_chunks, half, rpt),
        out_type=jax.ShapeDtypeStruct((_NC, n_pad, d), jnp.float32),
        mesh=plsc.VectorSubcoreMesh(core_axis_name="c", subcore_axis_name="s"),
        scratch_types=[
            pltpu.VMEM_SHARED((acc_rows, d), jnp.float32),
            pltpu.VMEM_SHARED((acc_rows, d), jnp.float32),
            pltpu.VMEM((ept,), jnp.int32),
            pltpu.VMEM((ept,), jnp.int32),
            pltpu.VMEM((_K,), jnp.int32),
            pltpu.VMEM((_K,), jnp.int32),
            pltpu.VMEM((_K,), jnp.int32),
            pltpu.VMEM((_K,), jnp.int32),
            pltpu.VMEM((_K, d), jnp.float32),
            pltpu.VMEM((_K, d), jnp.float32),
            pltpu.SemaphoreType.DMA,
            pltpu.SemaphoreType.DMA,
            pltpu.SemaphoreType.DMA,
            pltpu.SemaphoreType.DMA,
        ],
    )
    agg2 = sc_agg(src, dst, x, zeros)

    bn = 2048
    out = pl.pallas_call(
        _tc_linear_body,
        grid=(n_pad // bn,),
        in_specs=[
            pl.BlockSpec((_NC, bn, d), lambda i: (0, i, 0)),
            pl.BlockSpec((c_out, d), lambda i: (0, 0)),
            pl.BlockSpec((1, c_out), lambda i: (0, 0)),
        ],
        out_specs=pl.BlockSpec((bn, c_out), lambda i: (i, 0)),
        out_shape=jax.ShapeDtypeStruct((n_pad, c_out), jnp.float32),
    )(agg2, W, b.reshape(1, c_out))
    return out[:n]


# speedup vs baseline: 6.5536x; 1.0066x over previous
import functools

import jax
import jax.numpy as jnp
from jax import lax
from jax.experimental import pallas as pl
from jax.experimental.pallas import tpu as pltpu
from jax.experimental.pallas import tpu_sc as plsc

_NC = 2
_NS = 16
_NW = _NC * _NS
_K = 80
_L = 16
_TRASH = 8


_NB = 2


def _sc_body(n_chunks, half, rpt, src_hbm, dst_hbm, x_hbm, zeros_hbm,
             out_hbm, acc0, acc1, sall, dall, idx0s, idx1s, rows, gsems,
             ssems):
    c = lax.axis_index("c")
    s = lax.axis_index("s")
    wid = s * _NC + c
    ept = n_chunks * _K
    base = wid * ept

    pltpu.sync_copy(src_hbm.at[pl.ds(base, ept)], sall)
    pltpu.sync_copy(dst_hbm.at[pl.ds(base, ept)], dall)

    row0 = s * rpt
    pltpu.sync_copy(zeros_hbm.at[pl.ds(row0, rpt)], acc0.at[pl.ds(row0, rpt)])
    pltpu.sync_copy(zeros_hbm.at[pl.ds(row0, rpt)], acc1.at[pl.ds(row0, rpt)])
    plsc.subcore_barrier()

    def fire_gather(ch, rows, gsem):
        off = jnp.minimum(ch, n_chunks - 1) * _K
        pltpu.async_copy(x_hbm.at[sall.at[pl.ds(off, _K)]], rows, gsem)

    def wait_gather(rows, gsem):
        pltpu.make_async_copy(x_hbm.at[pl.ds(0, _K)], rows, gsem).wait()

    def fire_scatter(ch, rows, idx0, idx1, ssem):
        valid = jnp.where(ch < n_chunks, 1, 0).astype(jnp.int32)
        off = jnp.minimum(ch, n_chunks - 1) * _K
        for g in range(_K // _L):
            dv = dall[pl.ds(off + g * _L, _L)]
            lo = dv < half
            idx0[pl.ds(g * _L, _L)] = jnp.where(lo, dv + _TRASH, 0) * valid
            idx1[pl.ds(g * _L, _L)] = jnp.where(lo, 0, dv - half + _TRASH) * valid
        pltpu.async_copy(rows, acc0.at[idx0], ssem, add=True)
        pltpu.async_copy(rows, acc1.at[idx1], ssem, add=True)

    def wait_scatters(rows, ssem):
        pltpu.make_async_copy(x_hbm.at[pl.ds(0, _K)], rows, ssem).wait()
        pltpu.make_async_copy(x_hbm.at[pl.ds(0, _K)], rows, ssem).wait()

    for j in range(_NB):
        fire_gather(j, rows[j], gsems[j])

    def quad(o, _):
        c0 = _NB * o
        for j in range(_NB):
            wait_gather(rows[j], gsems[j])
            fire_scatter(c0 + j, rows[j], idx0s[j], idx1s[j], ssems[j])
        for j in range(_NB):
            wait_scatters(rows[j], ssems[j])
            fire_gather(c0 + _NB + j, rows[j], gsems[j])
        return 0

    n_groups = -(-n_chunks // _NB)
    lax.fori_loop(0, n_groups, quad, 0)
    for j in range(_NB):
        wait_gather(rows[j], gsems[j])

    plsc.subcore_barrier()
    wpt = half // _NS
    wrow = s * wpt
    pltpu.sync_copy(acc0.at[pl.ds(_TRASH + wrow, wpt)],
                    out_hbm.at[c, pl.ds(wrow, wpt)])
    pltpu.sync_copy(acc1.at[pl.ds(_TRASH + wrow, wpt)],
                    out_hbm.at[c, pl.ds(half + wrow, wpt)])


def _tc_linear_body(a_ref, w_ref, b_ref, o_ref):
    acc = a_ref[0] + a_ref[1]
    o_ref[...] = lax.dot_general(
        acc, w_ref[...], (((1,), (1,)), ((), ())),
        preferred_element_type=jnp.float32) + b_ref[...]


def kernel(x, edge_index, W, b):
    n, d = x.shape
    e = edge_index.shape[1]
    c_out = W.shape[0]
    assert e % (_NW * _K) == 0
    n_chunks = e // (_NW * _K)
    half = -(-n // (2 * _NS * 8)) * (_NS * 8)
    n_pad = 2 * half
    rpt = -(-(half + _TRASH) // (_NS * 8)) * 8
    acc_rows = rpt * _NS
    ept = n_chunks * _K

    src = edge_index[0]
    dst = edge_index[1]
    zeros = jnp.zeros((acc_rows, d), jnp.float32)

    sc_agg = pl.kernel(
        functools.partial(_sc_body, n_chunks, half, rpt),
        out_type=jax.ShapeDtypeStruct((_NC, n_pad, d), jnp.float32),
        mesh=plsc.VectorSubcoreMesh(core_axis_name="c", subcore_axis_name="s"),
        scratch_types=[
            pltpu.VMEM_SHARED((acc_rows, d), jnp.float32),
            pltpu.VMEM_SHARED((acc_rows, d), jnp.float32),
            pltpu.VMEM((ept,), jnp.int32),
            pltpu.VMEM((ept,), jnp.int32),
            [pltpu.VMEM((_K,), jnp.int32) for _ in range(_NB)],
            [pltpu.VMEM((_K,), jnp.int32) for _ in range(_NB)],
            [pltpu.VMEM((_K, d), jnp.float32) for _ in range(_NB)],
            [pltpu.SemaphoreType.DMA for _ in range(_NB)],
            [pltpu.SemaphoreType.DMA for _ in range(_NB)],
        ],
    )
    agg2 = sc_agg(src, dst, x, zeros)

    bn = 2048
    out = pl.pallas_call(
        _tc_linear_body,
        grid=(n_pad // bn,),
        in_specs=[
            pl.BlockSpec((_NC, bn, d), lambda i: (0, i, 0)),
            pl.BlockSpec((c_out, d), lambda i: (0, 0)),
            pl.BlockSpec((1, c_out), lambda i: (0, 0)),
        ],
        out_specs=pl.BlockSpec((bn, c_out), lambda i: (i, 0)),
        out_shape=jax.ShapeDtypeStruct((n_pad, c_out), jnp.float32),
    )(agg2, W, b.reshape(1, c_out))
    return out[:n]


# speedup vs baseline: 7.6279x; 1.1639x over previous
import functools

import jax
import jax.numpy as jnp
from jax import lax
from jax.experimental import pallas as pl
from jax.experimental.pallas import tpu as pltpu
from jax.experimental.pallas import tpu_sc as plsc

_NC = 2
_NS = 16
_NW = _NC * _NS
_K = 80
_L = 16
_TRASH = 16


_NB = 2


def _sc_body(n_chunks, half, rpt, src_hbm, dst_hbm, x_hbm, zeros_hbm,
             out_hbm, acc0, acc1, sall, dall, idx0s, idx1s, rows, gsems,
             ssems):
    c = lax.axis_index("c")
    s = lax.axis_index("s")
    wid = s * _NC + c
    ept = n_chunks * _K
    base = wid * ept

    pltpu.sync_copy(src_hbm.at[pl.ds(base, ept)], sall)
    pltpu.sync_copy(dst_hbm.at[pl.ds(base, ept)], dall)

    row0 = s * rpt
    pltpu.sync_copy(zeros_hbm.at[pl.ds(row0, rpt)], acc0.at[pl.ds(row0, rpt)])
    pltpu.sync_copy(zeros_hbm.at[pl.ds(row0, rpt)], acc1.at[pl.ds(row0, rpt)])
    plsc.subcore_barrier()

    def fire_gather(ch, rows, gsem):
        off = jnp.minimum(ch, n_chunks - 1) * _K
        pltpu.async_copy(x_hbm.at[sall.at[pl.ds(off, _K)]], rows, gsem)

    def wait_gather(rows, gsem):
        pltpu.make_async_copy(x_hbm.at[pl.ds(0, _K)], rows, gsem).wait()

    def fire_scatter(ch, rows, idx0, idx1, ssem):
        valid = jnp.where(ch < n_chunks, 1, 0).astype(jnp.int32)
        off = jnp.minimum(ch, n_chunks - 1) * _K
        tr = jax.lax.iota(jnp.int32, _L)
        for g in range(_K // _L):
            dv = dall[pl.ds(off + g * _L, _L)]
            lo = dv < half
            idx0[pl.ds(g * _L, _L)] = jnp.where(lo, dv + _TRASH, tr) * valid
            idx1[pl.ds(g * _L, _L)] = jnp.where(lo, tr, dv - half + _TRASH) * valid
        pltpu.async_copy(rows, acc0.at[idx0], ssem, add=True)
        pltpu.async_copy(rows, acc1.at[idx1], ssem, add=True)

    def wait_scatters(rows, ssem):
        pltpu.make_async_copy(x_hbm.at[pl.ds(0, _K)], rows, ssem).wait()
        pltpu.make_async_copy(x_hbm.at[pl.ds(0, _K)], rows, ssem).wait()

    for j in range(_NB):
        fire_gather(j, rows[j], gsems[j])

    def quad(o, _):
        c0 = _NB * o
        for j in range(_NB):
            wait_gather(rows[j], gsems[j])
            fire_scatter(c0 + j, rows[j], idx0s[j], idx1s[j], ssems[j])
        for j in range(_NB):
            wait_scatters(rows[j], ssems[j])
            fire_gather(c0 + _NB + j, rows[j], gsems[j])
        return 0

    n_groups = -(-n_chunks // _NB)
    lax.fori_loop(0, n_groups, quad, 0)
    for j in range(_NB):
        wait_gather(rows[j], gsems[j])

    plsc.subcore_barrier()
    wpt = half // _NS
    wrow = s * wpt
    pltpu.sync_copy(acc0.at[pl.ds(_TRASH + wrow, wpt)],
                    out_hbm.at[c, pl.ds(wrow, wpt)])
    pltpu.sync_copy(acc1.at[pl.ds(_TRASH + wrow, wpt)],
                    out_hbm.at[c, pl.ds(half + wrow, wpt)])


def _tc_linear_body(a_ref, w_ref, b_ref, o_ref):
    acc = a_ref[0] + a_ref[1]
    o_ref[...] = lax.dot_general(
        acc, w_ref[...], (((1,), (1,)), ((), ())),
        preferred_element_type=jnp.float32) + b_ref[...]


def kernel(x, edge_index, W, b):
    n, d = x.shape
    e = edge_index.shape[1]
    c_out = W.shape[0]
    assert e % (_NW * _K) == 0
    n_chunks = e // (_NW * _K)
    half = -(-n // (2 * _NS * 8)) * (_NS * 8)
    n_pad = 2 * half
    rpt = -(-(half + _TRASH) // (_NS * 8)) * 8
    acc_rows = rpt * _NS
    ept = n_chunks * _K

    src = edge_index[0]
    dst = edge_index[1]
    zeros = jnp.zeros((acc_rows, d), jnp.float32)

    sc_agg = pl.kernel(
        functools.partial(_sc_body, n_chunks, half, rpt),
        out_type=jax.ShapeDtypeStruct((_NC, n_pad, d), jnp.float32),
        mesh=plsc.VectorSubcoreMesh(core_axis_name="c", subcore_axis_name="s"),
        scratch_types=[
            pltpu.VMEM_SHARED((acc_rows, d), jnp.float32),
            pltpu.VMEM_SHARED((acc_rows, d), jnp.float32),
            pltpu.VMEM((ept,), jnp.int32),
            pltpu.VMEM((ept,), jnp.int32),
            [pltpu.VMEM((_K,), jnp.int32) for _ in range(_NB)],
            [pltpu.VMEM((_K,), jnp.int32) for _ in range(_NB)],
            [pltpu.VMEM((_K, d), jnp.float32) for _ in range(_NB)],
            [pltpu.SemaphoreType.DMA for _ in range(_NB)],
            [pltpu.SemaphoreType.DMA for _ in range(_NB)],
        ],
    )
    agg2 = sc_agg(src, dst, x, zeros)

    bn = 2048
    out = pl.pallas_call(
        _tc_linear_body,
        grid=(n_pad // bn,),
        in_specs=[
            pl.BlockSpec((_NC, bn, d), lambda i: (0, i, 0)),
            pl.BlockSpec((c_out, d), lambda i: (0, 0)),
            pl.BlockSpec((1, c_out), lambda i: (0, 0)),
        ],
        out_specs=pl.BlockSpec((bn, c_out), lambda i: (i, 0)),
        out_shape=jax.ShapeDtypeStruct((n_pad, c_out), jnp.float32),
    )(agg2, W, b.reshape(1, c_out))
    return out[:n]


# speedup vs baseline: 7.6363x; 1.0011x over previous
import functools

import jax
import jax.numpy as jnp
from jax import lax
from jax.experimental import pallas as pl
from jax.experimental.pallas import tpu as pltpu
from jax.experimental.pallas import tpu_sc as plsc

_NC = 2
_NS = 16
_NW = _NC * _NS
_K = 80
_L = 16
_TRASH = 64


_NB = 2


def _sc_body(n_chunks, half, rpt, src_hbm, dst_hbm, x_hbm, zeros_hbm,
             out_hbm, acc0, acc1, sall, dall, idx0s, idx1s, rows, gsems,
             ssems):
    c = lax.axis_index("c")
    s = lax.axis_index("s")
    wid = s * _NC + c
    ept = n_chunks * _K
    base = wid * ept

    pltpu.sync_copy(src_hbm.at[pl.ds(base, ept)], sall)
    pltpu.sync_copy(dst_hbm.at[pl.ds(base, ept)], dall)

    row0 = s * rpt
    pltpu.sync_copy(zeros_hbm.at[pl.ds(row0, rpt)], acc0.at[pl.ds(row0, rpt)])
    pltpu.sync_copy(zeros_hbm.at[pl.ds(row0, rpt)], acc1.at[pl.ds(row0, rpt)])
    plsc.subcore_barrier()

    def fire_gather(ch, rows, gsem):
        off = jnp.minimum(ch, n_chunks - 1) * _K
        pltpu.async_copy(x_hbm.at[sall.at[pl.ds(off, _K)]], rows, gsem)

    def wait_gather(rows, gsem):
        pltpu.make_async_copy(x_hbm.at[pl.ds(0, _K)], rows, gsem).wait()

    def compute_idx(ch, idx0, idx1):
        valid = jnp.where(ch < n_chunks, 1, 0).astype(jnp.int32)
        off = jnp.minimum(ch, n_chunks - 1) * _K
        for g in range(_K // _L):
            tr = jax.lax.iota(jnp.int32, _L) + (g % 4) * _L
            dv = dall[pl.ds(off + g * _L, _L)]
            lo = dv < half
            idx0[pl.ds(g * _L, _L)] = jnp.where(lo, dv + _TRASH, tr) * valid
            idx1[pl.ds(g * _L, _L)] = jnp.where(lo, tr, dv - half + _TRASH) * valid

    def fire_scatter(rows, idx0, idx1, ssem):
        pltpu.async_copy(rows, acc0.at[idx0], ssem, add=True)
        pltpu.async_copy(rows, acc1.at[idx1], ssem, add=True)

    def wait_scatters(rows, ssem):
        pltpu.make_async_copy(x_hbm.at[pl.ds(0, _K)], rows, ssem).wait()
        pltpu.make_async_copy(x_hbm.at[pl.ds(0, _K)], rows, ssem).wait()

    for j in range(_NB):
        fire_gather(j, rows[j], gsems[j])

    def quad(o, _):
        c0 = _NB * o
        for j in range(_NB):
            compute_idx(c0 + j, idx0s[j], idx1s[j])
        for j in range(_NB):
            wait_gather(rows[j], gsems[j])
            fire_scatter(rows[j], idx0s[j], idx1s[j], ssems[j])
        for j in range(_NB):
            wait_scatters(rows[j], ssems[j])
            fire_gather(c0 + _NB + j, rows[j], gsems[j])
        return 0

    n_groups = -(-n_chunks // _NB)
    lax.fori_loop(0, n_groups, quad, 0)
    for j in range(_NB):
        wait_gather(rows[j], gsems[j])

    plsc.subcore_barrier()
    wpt = half // _NS
    wrow = s * wpt
    pltpu.sync_copy(acc0.at[pl.ds(_TRASH + wrow, wpt)],
                    out_hbm.at[c, pl.ds(wrow, wpt)])
    pltpu.sync_copy(acc1.at[pl.ds(_TRASH + wrow, wpt)],
                    out_hbm.at[c, pl.ds(half + wrow, wpt)])


def _tc_linear_body(a_ref, w_ref, b_ref, o_ref):
    acc = a_ref[0] + a_ref[1]
    o_ref[...] = lax.dot_general(
        acc, w_ref[...], (((1,), (1,)), ((), ())),
        preferred_element_type=jnp.float32) + b_ref[...]


def kernel(x, edge_index, W, b):
    n, d = x.shape
    e = edge_index.shape[1]
    c_out = W.shape[0]
    assert e % (_NW * _K) == 0
    n_chunks = e // (_NW * _K)
    half = -(-n // (2 * _NS * 8)) * (_NS * 8)
    n_pad = 2 * half
    rpt = -(-(half + _TRASH) // (_NS * 8)) * 8
    acc_rows = rpt * _NS
    ept = n_chunks * _K

    src = edge_index[0]
    dst = edge_index[1]
    zeros = jnp.zeros((acc_rows, d), jnp.float32)

    sc_agg = pl.kernel(
        functools.partial(_sc_body, n_chunks, half, rpt),
        out_type=jax.ShapeDtypeStruct((_NC, n_pad, d), jnp.float32),
        mesh=plsc.VectorSubcoreMesh(core_axis_name="c", subcore_axis_name="s"),
        scratch_types=[
            pltpu.VMEM_SHARED((acc_rows, d), jnp.float32),
            pltpu.VMEM_SHARED((acc_rows, d), jnp.float32),
            pltpu.VMEM((ept,), jnp.int32),
            pltpu.VMEM((ept,), jnp.int32),
            [pltpu.VMEM((_K,), jnp.int32) for _ in range(_NB)],
            [pltpu.VMEM((_K,), jnp.int32) for _ in range(_NB)],
            [pltpu.VMEM((_K, d), jnp.float32) for _ in range(_NB)],
            [pltpu.SemaphoreType.DMA for _ in range(_NB)],
            [pltpu.SemaphoreType.DMA for _ in range(_NB)],
        ],
    )
    agg2 = sc_agg(src, dst, x, zeros)

    bn = 2048
    out = pl.pallas_call(
        _tc_linear_body,
        grid=(n_pad // bn,),
        in_specs=[
            pl.BlockSpec((_NC, bn, d), lambda i: (0, i, 0)),
            pl.BlockSpec((c_out, d), lambda i: (0, 0)),
            pl.BlockSpec((1, c_out), lambda i: (0, 0)),
        ],
        out_specs=pl.BlockSpec((bn, c_out), lambda i: (i, 0)),
        out_shape=jax.ShapeDtypeStruct((n_pad, c_out), jnp.float32),
    )(agg2, W, b.reshape(1, c_out))
    return out[:n]
